# baseline (speedup 1.0000x reference)
import functools

import jax
import jax.numpy as jnp
from jax import lax
from jax.experimental import pallas as pl
from jax.experimental.pallas import tpu as pltpu
from jax.experimental.pallas import tpu_sc as plsc

HIDDEN = 768
EPS = 1e-12

NC = 2
NS = 16
NW = NC * NS

TOKENS = 8192
N_SPLIT = 4
SPLIT = TOKENS // N_SPLIT
B_PER_W = SPLIT // NW
CHUNK = 64
N_CHUNKS = B_PER_W // CHUNK

TOK_BLK = 1024


def _sc_gather(table, ids, tok_base):
    mesh = plsc.VectorSubcoreMesh(core_axis_name="c", subcore_axis_name="s")

    @functools.partial(
        pl.kernel,
        mesh=mesh,
        out_type=jax.ShapeDtypeStruct((SPLIT, HIDDEN), jnp.float32),
        scratch_types=[
            pltpu.VMEM((B_PER_W,), jnp.int32),
            pltpu.VMEM((CHUNK, HIDDEN), jnp.float32),
            pltpu.VMEM((CHUNK, HIDDEN), jnp.float32),
            pltpu.SemaphoreType.DMA,
            pltpu.SemaphoreType.DMA,
            pltpu.SemaphoreType.DMA,
            pltpu.SemaphoreType.DMA,
        ],
    )
    def k(table_hbm, idx_hbm, out_hbm, idx_v, rows_a, rows_b, g0, g1, s0, s1):
        wid = lax.axis_index("s") * NC + lax.axis_index("c")
        base = wid * B_PER_W
        pltpu.sync_copy(idx_hbm.at[pl.ds(tok_base + base, B_PER_W)], idx_v)

        bufs = (rows_a, rows_b)
        gsems = (g0, g1)
        ssems = (s0, s1)
        gathers = [None, None]
        stores = [None, None]
        for c in range(N_CHUNKS):
            p = c % 2
            if stores[p] is not None:
                stores[p].wait()
            gathers[p] = pltpu.async_copy(
                table_hbm.at[idx_v.at[pl.ds(c * CHUNK, CHUNK)]], bufs[p], gsems[p]
            )
            if c >= 1:
                q = (c - 1) % 2
                gathers[q].wait()
                stores[q] = pltpu.async_copy(
                    bufs[q], out_hbm.at[pl.ds(base + (c - 1) * CHUNK, CHUNK)], ssems[q]
                )
        last = (N_CHUNKS - 1) % 2
        gathers[last].wait()
        stores[last] = pltpu.async_copy(
            bufs[last], out_hbm.at[pl.ds(base + (N_CHUNKS - 1) * CHUNK, CHUNK)],
            ssems[last],
        )
        if N_CHUNKS > 1:
            stores[1 - last].wait()
        stores[last].wait()

    return k(table, ids)


def _ln_math(x_ref, pos_ref, g_ref, b_ref, o_ref):
    x = x_ref[...] + pos_ref[...]
    mean = jnp.mean(x, axis=1, keepdims=True)
    xc = x - mean
    var = jnp.mean(xc * xc, axis=1, keepdims=True)
    inv = lax.rsqrt(var + EPS)
    o_ref[...] = xc * inv * g_ref[...] + b_ref[...]


def _ln_body_first(x_ref, pos_ref, g_ref, b_ref, o_ref):
    _ln_math(x_ref, pos_ref, g_ref, b_ref, o_ref)


def _ln_body_alias(x_ref, pos_ref, g_ref, b_ref, prev_ref, o_ref):
    del prev_ref
    _ln_math(x_ref, pos_ref, g_ref, b_ref, o_ref)


def _tc_ln_half(gathered, pos, gamma, beta, batch_half, seq_len, blk_base, prev):
    bps = seq_len // TOK_BLK
    in_specs = [
        pl.BlockSpec((TOK_BLK, HIDDEN), lambda i, j: (j * bps + i, 0)),
        pl.BlockSpec((TOK_BLK, HIDDEN), lambda i, j: (i, 0)),
        pl.BlockSpec((1, HIDDEN), lambda i, j: (0, 0)),
        pl.BlockSpec((1, HIDDEN), lambda i, j: (0, 0)),
    ]
    args = [gathered, pos, gamma.reshape(1, HIDDEN), beta.reshape(1, HIDDEN)]
    kwargs = {}
    if prev is None:
        body = _ln_body_first
    else:
        body = _ln_body_alias
        in_specs.append(pl.BlockSpec(memory_space=pl.ANY))
        args.append(prev)
        kwargs["input_output_aliases"] = {4: 0}
    return pl.pallas_call(
        body,
        grid=(bps, batch_half),
        in_specs=in_specs,
        out_specs=pl.BlockSpec(
            (TOK_BLK, HIDDEN), lambda i, j: (blk_base + j * bps + i, 0)
        ),
        out_shape=jax.ShapeDtypeStruct((TOKENS, HIDDEN), jnp.float32),
        **kwargs,
    )(*args)


def kernel(input_ids, word_embeddings, position_embeddings, ln_gamma, ln_beta):
    batch, seq = input_ids.shape
    assert batch * seq == TOKENS
    assert batch % N_SPLIT == 0
    batch_half = batch // N_SPLIT
    ids = input_ids.reshape(-1).astype(jnp.int32)

    gathered = [
        _sc_gather(word_embeddings, ids, h * SPLIT) for h in range(N_SPLIT)
    ]
    out = None
    for h in range(N_SPLIT):
        out = _tc_ln_half(
            gathered[h], position_embeddings, ln_gamma, ln_beta,
            batch_half, seq, h * (SPLIT // TOK_BLK), out,
        )
    return out.reshape(batch, seq, HIDDEN)


# speedup vs baseline: 1.1339x; 1.1339x over previous
import functools

import jax
import jax.numpy as jnp
from jax import lax
from jax.experimental import pallas as pl
from jax.experimental.pallas import tpu as pltpu
from jax.experimental.pallas import tpu_sc as plsc

HIDDEN = 768
EPS = 1e-12

NC = 2
NS = 16
NW = NC * NS

TOKENS = 8192
N_SPLIT = 2
SPLIT = TOKENS // N_SPLIT
B_PER_W = SPLIT // NW
CHUNK = 32
N_CHUNKS = B_PER_W // CHUNK

TOK_BLK = 2048


def _sc_gather(table, ids2d, tok_base, seq):
    mesh = plsc.VectorSubcoreMesh(core_axis_name="c", subcore_axis_name="s")
    w_per_row = seq // B_PER_W

    @functools.partial(
        pl.kernel,
        mesh=mesh,
        out_type=jax.ShapeDtypeStruct((SPLIT, HIDDEN), jnp.float32),
        scratch_types=[
            pltpu.VMEM((B_PER_W,), jnp.int32),
            pltpu.VMEM((N_CHUNKS, CHUNK, HIDDEN), jnp.float32),
            pltpu.SemaphoreType.DMA,
            pltpu.SemaphoreType.DMA,
        ],
    )
    def k(table_hbm, idx_hbm, out_hbm, idx_v, rows_v, gsem, ssem):
        wid = lax.axis_index("s") * NC + lax.axis_index("c")
        tok = tok_base + wid * B_PER_W
        pltpu.sync_copy(
            idx_hbm.at[tok // seq, pl.ds((wid % w_per_row) * B_PER_W, B_PER_W)],
            idx_v,
        )
        gathers = []
        for c in range(N_CHUNKS):
            gathers.append(pltpu.async_copy(
                table_hbm.at[idx_v.at[pl.ds(c * CHUNK, CHUNK)]],
                rows_v.at[c], gsem,
            ))
        stores = []
        for c in range(N_CHUNKS):
            gathers[c].wait()
            stores.append(pltpu.async_copy(
                rows_v.at[c],
                out_hbm.at[pl.ds(wid * B_PER_W + c * CHUNK, CHUNK)], ssem,
            ))
        for st in stores:
            st.wait()

    return k(table, ids2d)


def _ln_math(x_ref, pos_ref, g_ref, b_ref, o_ref):
    x = x_ref[...] + pos_ref[...]
    mean = jnp.mean(x, axis=1, keepdims=True)
    xc = x - mean
    var = jnp.mean(xc * xc, axis=1, keepdims=True)
    inv = lax.rsqrt(var + EPS)
    o_ref[...] = xc * inv * g_ref[...] + b_ref[...]


def _ln_body_first(x_ref, pos_ref, g_ref, b_ref, o_ref):
    _ln_math(x_ref, pos_ref, g_ref, b_ref, o_ref)


def _ln_body_alias(x_ref, pos_ref, g_ref, b_ref, prev_ref, o_ref):
    del prev_ref
    _ln_math(x_ref, pos_ref, g_ref, b_ref, o_ref)


def _tc_ln_half(gathered, pos, gamma, beta, batch_half, seq_len, blk_base, prev):
    bps = seq_len // TOK_BLK
    in_specs = [
        pl.BlockSpec((TOK_BLK, HIDDEN), lambda i, j: (j * bps + i, 0)),
        pl.BlockSpec((TOK_BLK, HIDDEN), lambda i, j: (i, 0)),
        pl.BlockSpec((1, HIDDEN), lambda i, j: (0, 0)),
        pl.BlockSpec((1, HIDDEN), lambda i, j: (0, 0)),
    ]
    args = [gathered, pos, gamma.reshape(1, HIDDEN), beta.reshape(1, HIDDEN)]
    kwargs = {}
    if prev is None:
        body = _ln_body_first
    else:
        body = _ln_body_alias
        in_specs.append(pl.BlockSpec(memory_space=pl.ANY))
        args.append(prev)
        kwargs["input_output_aliases"] = {4: 0}
    return pl.pallas_call(
        body,
        grid=(bps, batch_half),
        in_specs=in_specs,
        out_specs=pl.BlockSpec(
            (TOK_BLK, HIDDEN), lambda i, j: (blk_base + j * bps + i, 0)
        ),
        out_shape=jax.ShapeDtypeStruct((TOKENS, HIDDEN), jnp.float32),
        **kwargs,
    )(*args)


def kernel(input_ids, word_embeddings, position_embeddings, ln_gamma, ln_beta):
    batch, seq = input_ids.shape
    assert batch * seq == TOKENS
    assert batch % N_SPLIT == 0 and seq % B_PER_W == 0
    batch_half = batch // N_SPLIT
    ids2d = input_ids.astype(jnp.int32)

    gathered = [
        _sc_gather(word_embeddings, ids2d, h * SPLIT, seq) for h in range(N_SPLIT)
    ]
    out = None
    for h in range(N_SPLIT):
        out = _tc_ln_half(
            gathered[h], position_embeddings, ln_gamma, ln_beta,
            batch_half, seq, h * (SPLIT // TOK_BLK), out,
        )
    return out.reshape(batch, seq, HIDDEN)
